# baseline (speedup 1.0000x reference)
import math

import jax
import jax.numpy as jnp
from jax.experimental import pallas as pl

TOP_K = 8


def _router_body(x_ref, w_ref, n_ref, tkw_ref, tki_ref, g_ref, l_ref, lp_ref):
    bt, num_experts = l_ref.shape
    sigma = 1.0 / num_experts

    logits = jax.lax.dot_general(
        x_ref[...], w_ref[...], (((1,), (1,)), ((), ())),
        preferred_element_type=jnp.float32,
    )
    l_ref[...] = logits

    m = jnp.max(logits, axis=1, keepdims=True)
    e = jnp.exp(logits - m)
    s = jnp.sum(e, axis=1, keepdims=True)
    g_ref[...] = e / s

    rev_iota_f = (
        jnp.int32(num_experts - 1)
        - jax.lax.broadcasted_iota(jnp.int32, (bt, num_experts), 1)
    ).astype(jnp.float32)

    vals = e
    maxes = []
    ridxs = []
    for _ in range(TOP_K):
        mj = jnp.max(vals, axis=1, keepdims=True)
        hit = vals == mj
        rij = jnp.max(jnp.where(hit, rev_iota_f, -jnp.inf), axis=1, keepdims=True)
        maxes.append(mj)
        ridxs.append(rij)
        vals = jnp.where(hit & (rev_iota_f == rij), -jnp.inf, vals)
    e8 = jnp.concatenate(maxes, axis=1)
    r8 = jnp.concatenate(ridxs, axis=1)
    tki_ref[...] = jnp.int32(num_experts - 1) - r8.astype(jnp.int32)
    tkw_ref[...] = e8 / (jnp.sum(e8, axis=1, keepdims=True) + s * 1e-09)

    nv = logits + n_ref[...] * sigma
    for _ in range(TOP_K - 1):
        mj = jnp.max(nv, axis=1, keepdims=True)
        hit = nv == mj
        rij = jnp.max(jnp.where(hit, rev_iota_f, -jnp.inf), axis=1, keepdims=True)
        nv = jnp.where(hit & (rev_iota_f == rij), -jnp.inf, nv)
    tau = jnp.max(nv, axis=1, keepdims=True)

    z = (tau - logits) * num_experts
    lp_ref[...] = 0.5 * (1.0 - jax.lax.erf(z * (1.0 / math.sqrt(2.0))))


def kernel(x, W, noise):
    n_tokens, hidden = x.shape
    num_experts = W.shape[0]
    bt = 1024
    grid = (n_tokens // bt,)

    out_shapes = (
        jax.ShapeDtypeStruct((n_tokens, TOP_K), jnp.float32),
        jax.ShapeDtypeStruct((n_tokens, TOP_K), jnp.int32),
        jax.ShapeDtypeStruct((n_tokens, num_experts), jnp.float32),
        jax.ShapeDtypeStruct((n_tokens, num_experts), jnp.float32),
        jax.ShapeDtypeStruct((n_tokens, num_experts), jnp.float32),
    )
    row_spec = lambda d: pl.BlockSpec((bt, d), lambda i: (i, 0))
    tkw, tki, g, logits, lp = pl.pallas_call(
        _router_body,
        grid=grid,
        in_specs=[
            row_spec(hidden),
            pl.BlockSpec((num_experts, hidden), lambda i: (0, 0)),
            row_spec(num_experts),
        ],
        out_specs=(
            row_spec(TOP_K),
            row_spec(TOP_K),
            row_spec(num_experts),
            row_spec(num_experts),
            row_spec(num_experts),
        ),
        out_shape=out_shapes,
    )(x, W, noise)
    return (tkw, tki, g, logits, lp, tki)


# speedup vs baseline: 1.0905x; 1.0905x over previous
import math

import jax
import jax.numpy as jnp
from jax.experimental import pallas as pl

TOP_K = 8


def _pack_keys(v, iota, rev_iota):
    b = jax.lax.bitcast_convert_type(v, jnp.int32)
    low = jnp.where(b < 0, iota, rev_iota)
    return jax.lax.bitcast_convert_type((b & jnp.int32(~63)) | low, jnp.float32)


def _unpack(keys):
    b = jax.lax.bitcast_convert_type(keys, jnp.int32)
    low = b & jnp.int32(63)
    idx = jnp.where(b < 0, low, jnp.int32(63) - low)
    vals = jax.lax.bitcast_convert_type(b & jnp.int32(~63), jnp.float32)
    return vals, idx


def _router_body(x_ref, w_ref, n_ref, tkw_ref, tki_ref, g_ref, l_ref, lp_ref):
    bt, num_experts = l_ref.shape
    sigma = 1.0 / num_experts

    logits = jax.lax.dot_general(
        x_ref[...], w_ref[...], (((1,), (1,)), ((), ())),
        preferred_element_type=jnp.float32,
    )
    l_ref[...] = logits

    m = jnp.max(logits, axis=1, keepdims=True)
    e = jnp.exp(logits - m)
    s = jnp.sum(e, axis=1, keepdims=True)
    g_ref[...] = e / s

    iota = jax.lax.broadcasted_iota(jnp.int32, (bt, num_experts), 1)
    rev_iota = jnp.int32(num_experts - 1) - iota

    keys = _pack_keys(logits, iota, rev_iota)
    maxes = []
    for _ in range(TOP_K):
        mj = jnp.max(keys, axis=1, keepdims=True)
        maxes.append(mj)
        keys = jnp.where(keys == mj, -jnp.inf, keys)
    k8 = jnp.concatenate(maxes, axis=1)
    v8, i8 = _unpack(k8)
    tki_ref[...] = i8
    e8 = jnp.exp(v8 - m)
    tkw_ref[...] = e8 / (jnp.sum(e8, axis=1, keepdims=True) + s * 1e-09)

    nkeys = _pack_keys(logits + n_ref[...] * sigma, iota, rev_iota)
    for _ in range(TOP_K - 1):
        mj = jnp.max(nkeys, axis=1, keepdims=True)
        nkeys = jnp.where(nkeys == mj, -jnp.inf, nkeys)
    tau, _ = _unpack(jnp.max(nkeys, axis=1, keepdims=True))

    z = (tau - logits) * num_experts
    lp_ref[...] = 0.5 * (1.0 - jax.lax.erf(z * (1.0 / math.sqrt(2.0))))


def kernel(x, W, noise):
    n_tokens, hidden = x.shape
    num_experts = W.shape[0]
    bt = 1024
    grid = (n_tokens // bt,)

    out_shapes = (
        jax.ShapeDtypeStruct((n_tokens, TOP_K), jnp.float32),
        jax.ShapeDtypeStruct((n_tokens, TOP_K), jnp.int32),
        jax.ShapeDtypeStruct((n_tokens, num_experts), jnp.float32),
        jax.ShapeDtypeStruct((n_tokens, num_experts), jnp.float32),
        jax.ShapeDtypeStruct((n_tokens, num_experts), jnp.float32),
    )
    row_spec = lambda d: pl.BlockSpec((bt, d), lambda i: (i, 0))
    tkw, tki, g, logits, lp = pl.pallas_call(
        _router_body,
        grid=grid,
        in_specs=[
            row_spec(hidden),
            pl.BlockSpec((num_experts, hidden), lambda i: (0, 0)),
            row_spec(num_experts),
        ],
        out_specs=(
            row_spec(TOP_K),
            row_spec(TOP_K),
            row_spec(num_experts),
            row_spec(num_experts),
            row_spec(num_experts),
        ),
        out_shape=out_shapes,
    )(x, W, noise)
    return (tkw, tki, g, logits, lp, tki)


# speedup vs baseline: 1.1279x; 1.0344x over previous
import math

import jax
import jax.numpy as jnp
from jax.experimental import pallas as pl

TOP_K = 8


def _pack_keys(v, iota, rev_iota):
    b = jax.lax.bitcast_convert_type(v, jnp.int32)
    low = jnp.where(b < 0, iota, rev_iota)
    return jax.lax.bitcast_convert_type((b & jnp.int32(~63)) | low, jnp.float32)


def _unpack(keys):
    b = jax.lax.bitcast_convert_type(keys, jnp.int32)
    low = b & jnp.int32(63)
    idx = jnp.where(b < 0, low, jnp.int32(63) - low)
    vals = jax.lax.bitcast_convert_type(b & jnp.int32(~63), jnp.float32)
    return vals, idx


def _router_body(*refs):
    n_split = (len(refs) - 6) // 2
    x_refs = refs[:n_split]
    w_refs = refs[n_split : 2 * n_split]
    n_ref, tkw_ref, tki_ref, g_ref, l_ref, lp_ref = refs[2 * n_split :]
    bt, num_experts = l_ref.shape
    sigma = 1.0 / num_experts

    logits = jax.lax.dot_general(
        x_refs[0][...], w_refs[0][...], (((1,), (1,)), ((), ())),
        preferred_element_type=jnp.float32,
    )
    for xr, wr in zip(x_refs[1:], w_refs[1:]):
        logits = logits + jax.lax.dot_general(
            xr[...], wr[...], (((1,), (1,)), ((), ())),
            preferred_element_type=jnp.float32,
        )
    l_ref[...] = logits

    m = jnp.max(logits, axis=1, keepdims=True)
    e = jnp.exp(logits - m)
    s = jnp.sum(e, axis=1, keepdims=True)
    g_ref[...] = e / s

    iota = jax.lax.broadcasted_iota(jnp.int32, (bt, num_experts), 1)
    rev_iota = jnp.int32(num_experts - 1) - iota

    keys = _pack_keys(logits, iota, rev_iota)
    maxes = []
    for _ in range(TOP_K):
        mj = jnp.max(keys, axis=1, keepdims=True)
        maxes.append(mj)
        keys = jnp.where(keys == mj, -jnp.inf, keys)
    k8 = jnp.concatenate(maxes, axis=1)
    v8, i8 = _unpack(k8)
    tki_ref[...] = i8
    e8 = jnp.exp(v8 - m)
    tkw_ref[...] = e8 / (jnp.sum(e8, axis=1, keepdims=True) + s * 1e-09)

    nkeys = _pack_keys(logits + n_ref[...] * sigma, iota, rev_iota)
    for _ in range(TOP_K - 1):
        mj = jnp.max(nkeys, axis=1, keepdims=True)
        nkeys = jnp.where(nkeys == mj, -jnp.inf, nkeys)
    tau, _ = _unpack(jnp.max(nkeys, axis=1, keepdims=True))

    z = (tau - logits) * num_experts
    lp_ref[...] = 0.5 * (1.0 - jax.lax.erf(z * (1.0 / math.sqrt(2.0))))


def kernel(x, W, noise):
    n_tokens, hidden = x.shape
    num_experts = W.shape[0]
    bt = 1024
    grid = (n_tokens // bt,)

    out_shapes = (
        jax.ShapeDtypeStruct((n_tokens, TOP_K), jnp.float32),
        jax.ShapeDtypeStruct((n_tokens, TOP_K), jnp.int32),
        jax.ShapeDtypeStruct((n_tokens, num_experts), jnp.float32),
        jax.ShapeDtypeStruct((n_tokens, num_experts), jnp.float32),
        jax.ShapeDtypeStruct((n_tokens, num_experts), jnp.float32),
    )
    n_split = 2
    kc = hidden // n_split
    row_spec = lambda d: pl.BlockSpec((bt, d), lambda i: (i, 0))
    x_specs = [
        pl.BlockSpec((bt, kc), lambda i, j=j: (i, j)) for j in range(n_split)
    ]
    w_specs = [
        pl.BlockSpec((num_experts, kc), lambda i, j=j: (0, j))
        for j in range(n_split)
    ]
    tkw, tki, g, logits, lp = pl.pallas_call(
        _router_body,
        grid=grid,
        in_specs=x_specs + w_specs + [row_spec(num_experts)],
        out_specs=(
            row_spec(TOP_K),
            row_spec(TOP_K),
            row_spec(num_experts),
            row_spec(num_experts),
            row_spec(num_experts),
        ),
        out_shape=out_shapes,
    )(*([x] * n_split + [W] * n_split + [noise]))
    return (tkw, tki, g, logits, lp, tki)
